# Initial kernel scaffold; baseline (speedup 1.0000x reference)
import jax
import jax.numpy as jnp
from jax.experimental import pallas as pl


def kernel(inputs, interp_coe):
    raise NotImplementedError("write your pallas kernel here")


# speedup vs baseline: 1.3065x; 1.3065x over previous
import functools

import jax
import jax.numpy as jnp
from jax import lax
from jax.experimental import pallas as pl
from jax.experimental.pallas import tpu as pltpu
from jax.experimental.pallas import tpu_sc as plsc

MESH = 128
GRID = 2 * MESH + 1
GRID2 = GRID * GRID
NC, NS, L = 2, 16, 16
NW = NC * NS
B = 1024

OFFS = [kx * GRID2 + ky * GRID + kz
        for kx in range(3) for ky in range(3) for kz in range(3)]


def _sc_interp(xs, ys, zs, table, n):
    chunk = n // NW
    nsub = chunk // B
    mesh = plsc.VectorSubcoreMesh(core_axis_name="c", subcore_axis_name="s")

    @functools.partial(
        pl.kernel,
        mesh=mesh,
        out_type=jax.ShapeDtypeStruct((n,), jnp.float32),
        scratch_types=[
            pltpu.VMEM((3 * B,), jnp.float32),
            pltpu.VMEM((9 * B,), jnp.float32),
            pltpu.VMEM((27 * B,), jnp.int32),
            pltpu.VMEM((27 * B,), jnp.float32),
            pltpu.VMEM((B,), jnp.float32),
            pltpu.SemaphoreType.DMA,
        ],
    )
    def k(x_hbm, y_hbm, z_hbm, tab_hbm, out_hbm, xyz_v, bas_v, idx_v, val_v, acc_v, sem):
        wid = lax.axis_index("s") * NC + lax.axis_index("c")
        wbase = wid * chunk

        def subchunk(s, carry):
            base = wbase + s * B
            for dim, ref in enumerate((x_hbm, y_hbm, z_hbm)):
                pltpu.sync_copy(ref.at[pl.ds(base, B)],
                                xyz_v.at[pl.ds(dim * B, B)])

            def cvec(i, c2):
                o = pl.multiple_of(i * L, L)
                bidx = None
                for dim in range(3):
                    xn = jnp.clip(xyz_v[pl.ds(dim * B + o, L)], 0.0, 1.0) * float(MESH)
                    c = jnp.minimum(xn.astype(jnp.int32), MESH - 1)
                    t = xn - c.astype(jnp.float32)
                    bas_v[pl.ds((3 * dim + 0) * B + o, L)] = (2.0 * t - 1.0) * (t - 1.0)
                    bas_v[pl.ds((3 * dim + 1) * B + o, L)] = 4.0 * t * (1.0 - t)
                    bas_v[pl.ds((3 * dim + 2) * B + o, L)] = t * (2.0 * t - 1.0)
                    bidx = c if dim == 0 else bidx * GRID + c
                bidx = bidx * 2
                for kk in range(27):
                    idx_v[pl.ds(kk * B + o, L)] = bidx + OFFS[kk]
                return c2

            lax.fori_loop(0, B // L, cvec, 0, unroll=False)

            copies = [
                pltpu.make_async_copy(tab_hbm.at[idx_v.at[pl.ds(kk * B, B)]],
                                      val_v.at[pl.ds(kk * B, B)], sem)
                for kk in range(27)
            ]
            for cp in copies:
                cp.start()
            for cp in copies:
                cp.wait()

            def avec(i, c2):
                o = pl.multiple_of(i * L, L)
                bs = [bas_v[pl.ds(r * B + o, L)] for r in range(9)]
                acc = None
                for kk in range(27):
                    kx, ky, kz = kk // 9, (kk // 3) % 3, kk % 3
                    w = bs[kx] * bs[3 + ky] * bs[6 + kz]
                    term = w * val_v[pl.ds(kk * B + o, L)]
                    acc = term if acc is None else acc + term
                acc_v[pl.ds(o, L)] = acc
                return c2

            lax.fori_loop(0, B // L, avec, 0, unroll=False)
            pltpu.sync_copy(acc_v, out_hbm.at[pl.ds(base, B)])
            return carry

        lax.fori_loop(0, nsub, subchunk, 0, unroll=False)

    return k(xs, ys, zs, table)


def _mse(outputs, xt, n):
    rows = n // 128
    brows = 512
    grid = rows // brows
    o2 = outputs.reshape(rows, 128)
    x2 = xt.reshape(3, rows, 128)

    def body(x_ref, o_ref, out_ref):
        i = pl.program_id(0)
        x = x_ref[0]
        y = x_ref[1]
        z = x_ref[2]
        t = jnp.sin(x * 8.0) + jnp.cos(jnp.sqrt(y * 4.0)) * jnp.sin(z * 4.0)
        r = o_ref[...] - t

        @pl.when(i == 0)
        def _():
            out_ref[0, 0] = 0.0

        out_ref[0, 0] += jnp.sum(r * r)

    s = pl.pallas_call(
        body,
        grid=(grid,),
        in_specs=[
            pl.BlockSpec((3, brows, 128), lambda i: (0, i, 0)),
            pl.BlockSpec((brows, 128), lambda i: (i, 0)),
        ],
        out_specs=pl.BlockSpec((1, 1), lambda i: (0, 0), memory_space=pltpu.SMEM),
        out_shape=jax.ShapeDtypeStruct((1, 1), jnp.float32),
    )(x2, o2)
    return s[0, 0] / n


def kernel(inputs, interp_coe):
    n = inputs.shape[0]
    xt = inputs.T
    table = interp_coe.reshape(-1)
    o = _sc_interp(xt[0], xt[1], xt[2], table, n)
    return _mse(o, xt, n)


# speedup vs baseline: 2.3195x; 1.7754x over previous
import functools

import jax
import jax.numpy as jnp
from jax import lax
from jax.experimental import pallas as pl
from jax.experimental.pallas import tpu as pltpu
from jax.experimental.pallas import tpu_sc as plsc

MESH = 128
GRID = 2 * MESH + 1
NC, NS, L = 2, 16, 16
NW = NC * NS
B = 512

XS = 33 * 3 * 8 * 128
TYS = 3 * 8 * 128
TZS = 8 * 128


def _sc_interp(xs, ys, zs, table, n):
    chunk = n // NW
    nsub = chunk // B
    assert nsub % 2 == 0
    mesh = plsc.VectorSubcoreMesh(core_axis_name="c", subcore_axis_name="s")

    @functools.partial(
        pl.kernel,
        mesh=mesh,
        out_type=jax.ShapeDtypeStruct((n,), jnp.float32),
        scratch_types=[
            pltpu.VMEM((2 * 3 * B,), jnp.float32),
            pltpu.VMEM((2 * 9 * B,), jnp.float32),
            pltpu.VMEM((2 * 27 * B,), jnp.int32),
            pltpu.VMEM((2 * 27 * B,), jnp.float32),
            pltpu.VMEM((2 * B,), jnp.float32),
            pltpu.SemaphoreType.DMA,
            pltpu.SemaphoreType.DMA,
        ],
    )
    def k(x_hbm, y_hbm, z_hbm, tab_hbm, out_hbm, xyz_v, bas_v, idx_v, val_v,
          acc_v, sem0, sem1):
        wid = lax.axis_index("s") * NC + lax.axis_index("c")
        wbase = wid * chunk
        sems = (sem0, sem1)

        def stage_a(s, p):
            base = wbase + s * B
            xo, bo, io, vo = p * 3 * B, p * 9 * B, p * 27 * B, p * 27 * B
            for dim, ref in enumerate((x_hbm, y_hbm, z_hbm)):
                pltpu.sync_copy(ref.at[pl.ds(base, B)],
                                xyz_v.at[pl.ds(xo + dim * B, B)])

            def cvec(i, c2):
                o = pl.multiple_of(i * L, L)
                cells = []
                for dim in range(3):
                    xn = jnp.clip(xyz_v[pl.ds(xo + dim * B + o, L)], 0.0, 1.0) * float(MESH)
                    c = jnp.minimum(xn.astype(jnp.int32), MESH - 1)
                    t = xn - c.astype(jnp.float32)
                    bas_v[pl.ds(bo + (3 * dim + 0) * B + o, L)] = (2.0 * t - 1.0) * (t - 1.0)
                    bas_v[pl.ds(bo + (3 * dim + 1) * B + o, L)] = 4.0 * t * (1.0 - t)
                    bas_v[pl.ds(bo + (3 * dim + 2) * B + o, L)] = t * (2.0 * t - 1.0)
                    cells.append(c)
                cx, cy, cz = cells
                ax = [(cx * 2 + kq) * XS for kq in range(3)]
                by = []
                bz = []
                for kq in range(3):
                    g = cy * 2 + kq
                    by.append((g >> 3) * TYS + (g & 7) * 128)
                    g = cz * 2 + kq
                    bz.append((g >> 7) * TZS + (g & 127))
                for kx in range(3):
                    for ky in range(3):
                        axy = ax[kx] + by[ky]
                        for kz in range(3):
                            kk = (kx * 3 + ky) * 3 + kz
                            idx_v[pl.ds(io + kk * B + o, L)] = axy + bz[kz]
                return c2

            lax.fori_loop(0, B // L, cvec, 0, unroll=False)
            for kk in range(27):
                pltpu.make_async_copy(
                    tab_hbm.at[idx_v.at[pl.ds(io + kk * B, B)]],
                    val_v.at[pl.ds(vo + kk * B, B)], sems[p]).start()

        def stage_b(s, p):
            base = wbase + s * B
            bo, vo, ao = p * 9 * B, p * 27 * B, p * B
            for kk in range(27):
                pltpu.make_async_copy(
                    tab_hbm.at[idx_v.at[pl.ds(p * 27 * B + kk * B, B)]],
                    val_v.at[pl.ds(vo + kk * B, B)], sems[p]).wait()

            def avec(i, c2):
                o = pl.multiple_of(i * L, L)
                bs = [bas_v[pl.ds(bo + r * B + o, L)] for r in range(9)]
                acc = None
                for kk in range(27):
                    kx, ky, kz = kk // 9, (kk // 3) % 3, kk % 3
                    w = bs[kx] * bs[3 + ky] * bs[6 + kz]
                    term = w * val_v[pl.ds(vo + kk * B + o, L)]
                    acc = term if acc is None else acc + term
                acc_v[pl.ds(ao + o, L)] = acc
                return c2

            lax.fori_loop(0, B // L, avec, 0, unroll=False)
            pltpu.sync_copy(acc_v.at[pl.ds(ao, B)], out_hbm.at[pl.ds(base, B)])

        stage_a(0, 0)

        def outer(j, carry):
            s = 2 * j + 1
            stage_a(s, 1)
            stage_b(s - 1, 0)
            stage_a(s + 1, 0)
            stage_b(s, 1)
            return carry

        lax.fori_loop(0, nsub // 2 - 1, outer, 0, unroll=False)
        stage_a(nsub - 1, 1)
        stage_b(nsub - 2, 0)
        stage_b(nsub - 1, 1)

    return k(xs, ys, zs, table)


def _mse(outputs, xt, n):
    rows = n // 128
    brows = 512
    grid = rows // brows
    o2 = outputs.reshape(rows, 128)
    x2 = xt.reshape(3, rows, 128)

    def body(x_ref, o_ref, out_ref):
        i = pl.program_id(0)
        x = x_ref[0]
        y = x_ref[1]
        z = x_ref[2]
        t = jnp.sin(x * 8.0) + jnp.cos(jnp.sqrt(y * 4.0)) * jnp.sin(z * 4.0)
        r = o_ref[...] - t

        @pl.when(i == 0)
        def _():
            out_ref[0, 0] = 0.0

        out_ref[0, 0] += jnp.sum(r * r)

    s = pl.pallas_call(
        body,
        grid=(grid,),
        in_specs=[
            pl.BlockSpec((3, brows, 128), lambda i: (0, i, 0)),
            pl.BlockSpec((brows, 128), lambda i: (i, 0)),
        ],
        out_specs=pl.BlockSpec((1, 1), lambda i: (0, 0), memory_space=pltpu.SMEM),
        out_shape=jax.ShapeDtypeStruct((1, 1), jnp.float32),
    )(x2, o2)
    return s[0, 0] / n


def kernel(inputs, interp_coe):
    n = inputs.shape[0]
    xt = inputs.T
    cp = jnp.pad(interp_coe, ((0, 0), (0, 7), (0, 127)))
    table = cp.reshape(GRID, 33, 8, 3, 128).transpose(0, 1, 3, 2, 4).reshape(-1)
    o = _sc_interp(xt[0], xt[1], xt[2], table, n)
    return _mse(o, xt, n)
